# Initial kernel scaffold; baseline (speedup 1.0000x reference)
import jax
import jax.numpy as jnp
from jax.experimental import pallas as pl


def kernel(x, edge_index, batch, W1, b1, W2, b2, Wlin, blin):
    raise NotImplementedError("write your pallas kernel here")


# speedup vs baseline: 58.9211x; 58.9211x over previous
import functools

import jax
import jax.numpy as jnp
from jax import lax
from jax.experimental import pallas as pl
from jax.experimental.pallas import tpu as pltpu
from jax.experimental.pallas import tpu_sc as plsc

NC = 2
NS = 16
NW = NC * NS

N = 10000
E = 320000
G = 64

NPAD = 10240
RZ_DEG = NPAD // NS
RZ = NPAD // NS
EW = E // NW
EB = 2000


def _sc_mesh():
  return plsc.VectorSubcoreMesh(
      core_axis_name="c", subcore_axis_name="s", num_cores=NC,
      num_subcores=NS)


def _deg_kernel(dst_hbm, zeros_hbm, ones_hbm, out_hbm, dacc, idx_v, ones_v,
                zbuf):
  c = lax.axis_index("c")
  s = lax.axis_index("s")
  w = c * NS + s

  pltpu.sync_copy(zeros_hbm.at[pl.ds(s * RZ_DEG, RZ_DEG)], zbuf)
  pltpu.sync_copy(zbuf, dacc.at[pl.ds(s * RZ_DEG, RZ_DEG)])
  pltpu.sync_copy(ones_hbm, ones_v)
  plsc.subcore_barrier()

  for i in range(EW // EB):
    off = w * EW + i * EB
    pltpu.sync_copy(dst_hbm.at[pl.ds(off, EB)], idx_v)
    pltpu.sync_copy(ones_v, dacc.at[idx_v], add=True)

  plsc.subcore_barrier()
  pltpu.sync_copy(dacc.at[pl.ds(s * RZ_DEG, RZ_DEG)], zbuf)
  pltpu.sync_copy(zbuf, out_hbm.at[c, pl.ds(s * RZ_DEG, RZ_DEG)])


def _degrees(dst):
  kern = pl.kernel(
      _deg_kernel,
      out_type=jax.ShapeDtypeStruct((NC, NPAD), jnp.float32),
      mesh=_sc_mesh(),
      scratch_types=[
          pltpu.VMEM_SHARED((NPAD,), jnp.float32),
          pltpu.VMEM((EB,), jnp.int32),
          pltpu.VMEM((EB,), jnp.float32),
          pltpu.VMEM((RZ_DEG,), jnp.float32),
      ],
  )
  return kern(dst, jnp.zeros((NPAD,), jnp.float32),
              jnp.ones((EB,), jnp.float32))


def _agg_kernel(y_hbm, src_hbm, dst_hbm, zeros_hbm, out_hbm, acc, sidx, didx,
                rows, zrows):
  c = lax.axis_index("c")
  s = lax.axis_index("s")
  w = c * NS + s

  pltpu.sync_copy(zeros_hbm.at[pl.ds(s * RZ, RZ)], zrows)
  pltpu.sync_copy(zrows, acc.at[pl.ds(s * RZ, RZ)])
  plsc.subcore_barrier()

  for i in range(EW // EB):
    off = w * EW + i * EB
    pltpu.sync_copy(src_hbm.at[pl.ds(off, EB)], sidx)
    pltpu.sync_copy(dst_hbm.at[pl.ds(off, EB)], didx)
    pltpu.sync_copy(y_hbm.at[sidx], rows)
    pltpu.sync_copy(rows, acc.at[didx], add=True)

  plsc.subcore_barrier()
  pltpu.sync_copy(acc.at[pl.ds(s * RZ, RZ)], zrows)
  pltpu.sync_copy(zrows, out_hbm.at[c, pl.ds(s * RZ, RZ)])


def _aggregate(y, src, dst, h):
  kern = pl.kernel(
      _agg_kernel,
      out_type=jax.ShapeDtypeStruct((NC, NPAD, h), jnp.float32),
      mesh=_sc_mesh(),
      scratch_types=[
          pltpu.VMEM_SHARED((NPAD, h), jnp.float32),
          pltpu.VMEM((EB,), jnp.int32),
          pltpu.VMEM((EB,), jnp.int32),
          pltpu.VMEM((EB, h), jnp.float32),
          pltpu.VMEM((RZ, h), jnp.float32),
      ],
      compiler_params=pltpu.CompilerParams(use_tc_tiling_on_sc=False),
  )
  return kern(y, src, dst, jnp.zeros((NPAD, h), jnp.float32))


def _dinv_from(degt_ref):
  deg = degt_ref[:, 0:1] + degt_ref[:, 1:2] + 1.0
  return lax.rsqrt(deg)[:N]


def _tc1_body(degt_ref, x_ref, w1_ref, y1_ref):
  dinv = _dinv_from(degt_ref)
  y1_ref[:] = jnp.dot(x_ref[:], w1_ref[:],
                      preferred_element_type=jnp.float32) * dinv


def _tc2_body(degt_ref, y1_ref, agg1_ref, w2_ref, b1_ref, y2_ref):
  dinv = _dinv_from(degt_ref)
  agg = agg1_ref[0, :N] + agg1_ref[1, :N] + y1_ref[:]
  h1 = jax.nn.relu(dinv * agg + b1_ref[:])
  y2_ref[:] = jnp.dot(h1, w2_ref[:], preferred_element_type=jnp.float32) * dinv


def _tc3_body(degt_ref, y2_ref, agg2_ref, b2_ref, batch_ref, wlin_ref,
              blin_ref, out_ref):
  dinv = _dinv_from(degt_ref)
  agg = agg2_ref[0, :N] + agg2_ref[1, :N] + y2_ref[:]
  h2 = jax.nn.relu(dinv * agg + b2_ref[:])
  bt = batch_ref[:]
  gid = lax.broadcasted_iota(jnp.int32, (G, 1), 0)
  m = (bt == gid).astype(jnp.float32)
  ssum = jnp.dot(m, h2, preferred_element_type=jnp.float32)
  cnt = jnp.sum(m, axis=1, keepdims=True)
  pooled = ssum / jnp.maximum(cnt, 1.0)
  out_ref[:] = jnp.dot(pooled, wlin_ref[:],
                       preferred_element_type=jnp.float32) + blin_ref[:]


def kernel(x, edge_index, batch, W1, b1, W2, b2, Wlin, blin):
  src = edge_index[0]
  dst = edge_index[1]
  H1 = W1.shape[1]
  H2 = W2.shape[1]
  C = Wlin.shape[1]

  degp = _degrees(dst)
  degt = jnp.transpose(degp)

  y1 = pl.pallas_call(
      _tc1_body,
      out_shape=jax.ShapeDtypeStruct((N, H1), jnp.float32),
  )(degt, x, W1)

  agg1 = _aggregate(y1, src, dst, H1)

  y2 = pl.pallas_call(
      _tc2_body,
      out_shape=jax.ShapeDtypeStruct((N, H2), jnp.float32),
  )(degt, y1, agg1, W2, b1.reshape(1, H1))

  agg2 = _aggregate(y2, src, dst, H2)

  out = pl.pallas_call(
      _tc3_body,
      out_shape=jax.ShapeDtypeStruct((G, C), jnp.float32),
  )(degt, y2, agg2, b2.reshape(1, H2), batch.reshape(1, N),
    Wlin, blin.reshape(1, C))
  return out


# speedup vs baseline: 90.4649x; 1.5354x over previous
import jax
import jax.numpy as jnp
from jax import lax
from jax.experimental import pallas as pl
from jax.experimental.pallas import tpu as pltpu
from jax.experimental.pallas import tpu_sc as plsc

NC = 2
NS = 16
NW = NC * NS
L = 16

N = 10000
E = 320000
G = 64
HP = 16

NPAD = 10240
RZ = NPAD // NS
EW = E // NW
EB = 2000
RG = NPAD * HP // 128


def _sc_mesh():
  return plsc.VectorSubcoreMesh(
      core_axis_name="c", subcore_axis_name="s", num_cores=NC,
      num_subcores=NS)


def _deg_kernel(ei_hbm, ones_hbm, out_hbm, dacc, idx_v, ones_v, zrows):
  c = lax.axis_index("c")
  s = lax.axis_index("s")
  w = c * NS + s

  def _fill(i, carry):
    zrows[i, :] = jnp.zeros((L,), jnp.float32)
    return carry

  lax.fori_loop(0, RZ, _fill, 0)
  pltpu.sync_copy(zrows, dacc.at[pl.ds(s * RZ, RZ)])
  pltpu.sync_copy(ones_hbm, ones_v)
  plsc.subcore_barrier()

  for i in range(EW // EB):
    off = w * EW + i * EB
    pltpu.sync_copy(ei_hbm.at[1, pl.ds(off, EB)], idx_v)
    pltpu.sync_copy(ones_v, dacc.at[idx_v], add=True)

  plsc.subcore_barrier()
  pltpu.sync_copy(dacc.at[pl.ds(s * RZ, RZ)], zrows)
  pltpu.sync_copy(zrows, out_hbm.at[c, pl.ds(s * RZ, RZ)])


def _degrees(ei):
  kern = pl.kernel(
      _deg_kernel,
      out_type=jax.ShapeDtypeStruct((NC, NPAD, HP), jnp.float32),
      mesh=_sc_mesh(),
      scratch_types=[
          pltpu.VMEM_SHARED((NPAD, HP), jnp.float32),
          pltpu.VMEM((EB,), jnp.int32),
          pltpu.VMEM((EB, HP), jnp.float32),
          pltpu.VMEM((RZ, HP), jnp.float32),
      ],
      compiler_params=pltpu.CompilerParams(use_tc_tiling_on_sc=False),
  )
  return kern(ei, jnp.ones((EB, HP), jnp.float32))


def _agg_kernel(y_hbm, ei_hbm, out_hbm, ytab, acc, sidx, didx, rows, yrows):
  c = lax.axis_index("c")
  s = lax.axis_index("s")
  w = c * NS + s
  r0 = s * RZ

  pltpu.sync_copy(y_hbm.at[pl.ds(r0, RZ)], yrows)
  pltpu.sync_copy(yrows, ytab.at[pl.ds(r0, RZ)])

  @pl.when(c == 0)
  def _():
    pltpu.sync_copy(yrows, acc.at[pl.ds(r0, RZ)])

  @pl.when(c != 0)
  def _():
    def _fill(i, carry):
      rows[i, :] = jnp.zeros((L,), jnp.float32)
      return carry

    lax.fori_loop(0, RZ, _fill, 0)
    pltpu.sync_copy(rows.at[pl.ds(0, RZ)], acc.at[pl.ds(r0, RZ)])

  plsc.subcore_barrier()

  for i in range(EW // EB):
    off = w * EW + i * EB
    pltpu.sync_copy(ei_hbm.at[0, pl.ds(off, EB)], sidx)
    pltpu.sync_copy(ei_hbm.at[1, pl.ds(off, EB)], didx)
    pltpu.sync_copy(ytab.at[sidx], rows)
    pltpu.sync_copy(rows, acc.at[didx], add=True)

  plsc.subcore_barrier()
  pltpu.sync_copy(acc.at[pl.ds(r0, RZ)], yrows)
  pltpu.sync_copy(yrows, out_hbm.at[c, pl.ds(r0, RZ)])


def _aggregate(y, ei):
  kern = pl.kernel(
      _agg_kernel,
      out_type=jax.ShapeDtypeStruct((NC, NPAD, HP), jnp.float32),
      mesh=_sc_mesh(),
      scratch_types=[
          pltpu.VMEM_SHARED((NPAD, HP), jnp.float32),
          pltpu.VMEM_SHARED((NPAD, HP), jnp.float32),
          pltpu.VMEM((EB,), jnp.int32),
          pltpu.VMEM((EB,), jnp.int32),
          pltpu.VMEM((EB, HP), jnp.float32),
          pltpu.VMEM((RZ, HP), jnp.float32),
      ],
      compiler_params=pltpu.CompilerParams(use_tc_tiling_on_sc=False),
  )
  return kern(y, ei)


def _tc1_body(degr_ref, x_ref, w1_ref, y1_ref, dinv_ref):
  dinv = lax.rsqrt(degr_ref[0:RG] + degr_ref[RG:2 * RG] + 1.0)
  dinv_ref[:] = dinv
  xv = x_ref[:].reshape(N // 8, 8, 128)
  parts = [
      jnp.dot(xv[:, j, :], w1_ref[:], preferred_element_type=jnp.float32)
      for j in range(8)
  ]
  xwg = jnp.concatenate(parts, axis=1)
  y1_ref[0:N // 8, :] = xwg * dinv[0:N // 8]
  y1_ref[N // 8:, :] = jnp.zeros((RG - N // 8, 128), jnp.float32)


def _tc2_body(dinv_ref, agg1r_ref, w2bd_ref, b1g_ref, y2_ref):
  dinv = dinv_ref[:]
  sfull = agg1r_ref[0:RG] + agg1r_ref[RG:2 * RG]
  h1 = jax.nn.relu(dinv * sfull + b1g_ref[:])
  y2_ref[:] = jnp.dot(h1, w2bd_ref[:],
                      preferred_element_type=jnp.float32) * dinv


def _tc3_body(dinv_ref, agg2r_ref, b2g_ref, bt_ref, wlin_ref, blin_ref,
              out_ref):
  h2 = jax.nn.relu(
      dinv_ref[:] * (agg2r_ref[0:RG] + agg2r_ref[RG:2 * RG]) + b2g_ref[:])
  bt = bt_ref[:]
  gid = lax.broadcasted_iota(jnp.int32, (G, 1), 0)
  pooled = jnp.zeros((G, HP), jnp.float32)
  cnt = jnp.zeros((G, 1), jnp.float32)
  for j in range(8):
    mj = (bt[j:j + 1, :] == gid).astype(jnp.float32)
    pj = jnp.dot(mj, h2, preferred_element_type=jnp.float32)
    pooled = pooled + pj[:, HP * j:HP * (j + 1)]
    cnt = cnt + jnp.sum(mj, axis=1, keepdims=True)
  pooled = pooled / jnp.maximum(cnt, 1.0)
  out_ref[:] = jnp.dot(pooled, wlin_ref[:],
                       preferred_element_type=jnp.float32) + blin_ref[:]


def kernel(x, edge_index, batch, W1, b1, W2, b2, Wlin, blin):
  C = Wlin.shape[1]
  H1 = W1.shape[1]
  H2 = W2.shape[1]

  degp = _degrees(edge_index)

  y1g, dinvg = pl.pallas_call(
      _tc1_body,
      out_shape=[
          jax.ShapeDtypeStruct((RG, 128), jnp.float32),
          jax.ShapeDtypeStruct((RG, 128), jnp.float32),
      ],
  )(jnp.reshape(degp, (NC * RG, 128)), x, W1)

  agg1 = _aggregate(jnp.reshape(y1g, (NPAD, HP)), edge_index)

  w2bd = jnp.kron(jnp.eye(8, dtype=jnp.float32),
                  jnp.pad(W2, ((0, 0), (0, HP - H2))))
  b1g = jnp.tile(b1, 8).reshape(1, 128)
  y2g = pl.pallas_call(
      _tc2_body,
      out_shape=jax.ShapeDtypeStruct((RG, 128), jnp.float32),
  )(dinvg, jnp.reshape(agg1, (NC * RG, 128)), w2bd, b1g)

  agg2 = _aggregate(jnp.reshape(y2g, (NPAD, HP)), edge_index)

  b2g = jnp.tile(jnp.pad(b2, (0, HP - H2)), 8).reshape(1, 128)
  bt = jnp.concatenate(
      [batch, jnp.full((NPAD - N,), -1, jnp.int32)]).reshape(RG, 8).T
  wlinp = jnp.pad(Wlin, ((0, HP - H2), (0, 0)))
  out = pl.pallas_call(
      _tc3_body,
      out_shape=jax.ShapeDtypeStruct((G, C), jnp.float32),
  )(dinvg, jnp.reshape(agg2, (NC * RG, 128)), b2g, bt, wlinp,
    blin.reshape(1, C))
  return out


# speedup vs baseline: 103.6098x; 1.1453x over previous
import jax
import jax.numpy as jnp
from jax import lax
from jax.experimental import pallas as pl
from jax.experimental.pallas import tpu as pltpu
from jax.experimental.pallas import tpu_sc as plsc

NC = 2
NS = 16
NW = NC * NS
L = 16

N = 10000
E = 320000
G = 64
HP = 16

NPAD = 10240
RZ = NPAD // NS
EW = E // NW
EB = 2000
NCH = EW // EB
RG = NPAD * HP // 128


def _sc_mesh():
  return plsc.VectorSubcoreMesh(
      core_axis_name="c", subcore_axis_name="s", num_cores=NC,
      num_subcores=NS)


def _deg_kernel(ei_hbm, ones_hbm, out_hbm, dacc, idx_v, ones_v, zrows):
  c = lax.axis_index("c")
  s = lax.axis_index("s")
  w = c * NS + s

  def _fill(i, carry):
    zrows[i, :] = jnp.zeros((L,), jnp.float32)
    return carry

  lax.fori_loop(0, RZ, _fill, 0)
  pltpu.sync_copy(zrows, dacc.at[pl.ds(s * RZ, RZ)])
  pltpu.sync_copy(ones_hbm, ones_v)
  plsc.subcore_barrier()

  for i in range(NCH):
    off = w * EW + i * EB
    pltpu.sync_copy(ei_hbm.at[1, pl.ds(off, EB)], idx_v)
    pltpu.sync_copy(ones_v, dacc.at[idx_v], add=True)

  plsc.subcore_barrier()
  pltpu.sync_copy(dacc.at[pl.ds(s * RZ, RZ)], zrows)
  pltpu.sync_copy(zrows, out_hbm.at[c, pl.ds(s * RZ, RZ)])


def _degrees(ei):
  kern = pl.kernel(
      _deg_kernel,
      out_type=jax.ShapeDtypeStruct((NC, NPAD, HP), jnp.float32),
      mesh=_sc_mesh(),
      scratch_types=[
          pltpu.VMEM_SHARED((NPAD, HP), jnp.float32),
          pltpu.VMEM((EB,), jnp.int32),
          pltpu.VMEM((EB, HP), jnp.float32),
          pltpu.VMEM((RZ, HP), jnp.float32),
      ],
      compiler_params=pltpu.CompilerParams(use_tc_tiling_on_sc=False),
  )
  return kern(ei, jnp.ones((EB, HP), jnp.float32))


def _agg_kernel(y_hbm, ei_hbm, out_hbm, ytab, acc, sidx0, sidx1, didx0,
                didx1, rows0, rows1, yrows, semd0, semd1, semg0, semg1,
                sems0, sems1, semi0, semi1):
  c = lax.axis_index("c")
  s = lax.axis_index("s")
  w = c * NS + s
  r0 = s * RZ
  sidx = [sidx0, sidx1]
  didx = [didx0, didx1]
  rows = [rows0, rows1]
  semd = [semd0, semd1]
  semg = [semg0, semg1]
  sems = [sems0, sems1]
  semi = [semi0, semi1]

  pltpu.sync_copy(ei_hbm.at[0, pl.ds(w * EW, EB)], sidx0)
  pltpu.sync_copy(ei_hbm.at[1, pl.ds(w * EW, EB)], didx0)
  pltpu.sync_copy(y_hbm.at[pl.ds(r0, RZ)], yrows)
  pltpu.sync_copy(yrows, ytab.at[pl.ds(r0, RZ)])

  @pl.when(c == 0)
  def _():
    pltpu.sync_copy(yrows, acc.at[pl.ds(r0, RZ)])

  @pl.when(c != 0)
  def _():
    def _fill(i, carry):
      rows0[i, :] = jnp.zeros((L,), jnp.float32)
      return carry

    lax.fori_loop(0, RZ, _fill, 0)
    pltpu.sync_copy(rows0.at[pl.ds(0, RZ)], acc.at[pl.ds(r0, RZ)])

  plsc.subcore_barrier()

  gather_cp = [None] * NCH
  scatter_cp = [None] * NCH
  didx_cp = [None] * NCH
  sidx_cp = [None] * NCH
  gather_cp[0] = pltpu.async_copy(ytab.at[sidx0], rows[0], semg[0])
  for i in range(NCH):
    b = i % 2
    nb = (i + 1) % 2
    if i + 1 < NCH:
      if i > 0:
        scatter_cp[i - 1].wait()
      didx_cp[i + 1] = pltpu.async_copy(
          ei_hbm.at[1, pl.ds(w * EW + (i + 1) * EB, EB)], didx[nb], semd[nb])
      sidx_cp[i + 1] = pltpu.async_copy(
          ei_hbm.at[0, pl.ds(w * EW + (i + 1) * EB, EB)], sidx[nb], semi[nb])
    gather_cp[i].wait()
    if i > 0:
      didx_cp[i].wait()
    scatter_cp[i] = pltpu.async_copy(
        rows[b], acc.at[didx[b]], sems[b], add=True)
    if i + 1 < NCH:
      sidx_cp[i + 1].wait()
      gather_cp[i + 1] = pltpu.async_copy(
          ytab.at[sidx[nb]], rows[nb], semg[nb])
  scatter_cp[NCH - 2].wait()
  scatter_cp[NCH - 1].wait()

  plsc.subcore_barrier()
  pltpu.sync_copy(acc.at[pl.ds(r0, RZ)], yrows)
  pltpu.sync_copy(yrows, out_hbm.at[c, pl.ds(r0, RZ)])


def _aggregate(y, ei):
  kern = pl.kernel(
      _agg_kernel,
      out_type=jax.ShapeDtypeStruct((NC, NPAD, HP), jnp.float32),
      mesh=_sc_mesh(),
      scratch_types=[
          pltpu.VMEM_SHARED((NPAD, HP), jnp.float32),
          pltpu.VMEM_SHARED((NPAD, HP), jnp.float32),
          pltpu.VMEM((EB,), jnp.int32),
          pltpu.VMEM((EB,), jnp.int32),
          pltpu.VMEM((EB,), jnp.int32),
          pltpu.VMEM((EB,), jnp.int32),
          pltpu.VMEM((EB, HP), jnp.float32),
          pltpu.VMEM((EB, HP), jnp.float32),
          pltpu.VMEM((RZ, HP), jnp.float32),
          pltpu.SemaphoreType.DMA,
          pltpu.SemaphoreType.DMA,
          pltpu.SemaphoreType.DMA,
          pltpu.SemaphoreType.DMA,
          pltpu.SemaphoreType.DMA,
          pltpu.SemaphoreType.DMA,
          pltpu.SemaphoreType.DMA,
          pltpu.SemaphoreType.DMA,
      ],
      compiler_params=pltpu.CompilerParams(use_tc_tiling_on_sc=False),
  )
  return kern(y, ei)


def _tc1a_body(x_ref, w1_ref, xw_ref):
  xv = x_ref[:].reshape(N // 8, 8, 128)
  parts = [
      jnp.dot(xv[:, j, :], w1_ref[:], preferred_element_type=jnp.float32)
      for j in range(8)
  ]
  xw_ref[0:N // 8, :] = jnp.concatenate(parts, axis=1)
  xw_ref[N // 8:, :] = jnp.zeros((RG - N // 8, 128), jnp.float32)


def _tc1b_body(degr_ref, xw_ref, y1_ref, dinv_ref):
  dinv = lax.rsqrt(degr_ref[0:RG] + degr_ref[RG:2 * RG] + 1.0)
  dinv_ref[:] = dinv
  y1_ref[:] = xw_ref[:] * dinv


def _tc2_body(dinv_ref, agg1r_ref, w2bd_ref, b1g_ref, y2_ref):
  dinv = dinv_ref[:]
  sfull = agg1r_ref[0:RG] + agg1r_ref[RG:2 * RG]
  h1 = jax.nn.relu(dinv * sfull + b1g_ref[:])
  y2_ref[:] = jnp.dot(h1, w2bd_ref[:],
                      preferred_element_type=jnp.float32) * dinv


def _tc3_body(dinv_ref, agg2r_ref, b2g_ref, bt_ref, wlin_ref, blin_ref,
              out_ref):
  h2 = jax.nn.relu(
      dinv_ref[:] * (agg2r_ref[0:RG] + agg2r_ref[RG:2 * RG]) + b2g_ref[:])
  bt = bt_ref[:]
  gid = lax.broadcasted_iota(jnp.int32, (G, 1), 0)
  pooled = jnp.zeros((G, HP), jnp.float32)
  cnt = jnp.zeros((G, 1), jnp.float32)
  for j in range(8):
    mj = (bt[j:j + 1, :] == gid).astype(jnp.float32)
    pj = jnp.dot(mj, h2, preferred_element_type=jnp.float32)
    pooled = pooled + pj[:, HP * j:HP * (j + 1)]
    cnt = cnt + jnp.sum(mj, axis=1, keepdims=True)
  pooled = pooled / jnp.maximum(cnt, 1.0)
  out_ref[:] = jnp.dot(pooled, wlin_ref[:],
                       preferred_element_type=jnp.float32) + blin_ref[:]


def kernel(x, edge_index, batch, W1, b1, W2, b2, Wlin, blin):
  C = Wlin.shape[1]
  H2 = W2.shape[1]

  degp = _degrees(edge_index)

  xwg = pl.pallas_call(
      _tc1a_body,
      out_shape=jax.ShapeDtypeStruct((RG, 128), jnp.float32),
  )(x, W1)

  y1g, dinvg = pl.pallas_call(
      _tc1b_body,
      out_shape=[
          jax.ShapeDtypeStruct((RG, 128), jnp.float32),
          jax.ShapeDtypeStruct((RG, 128), jnp.float32),
      ],
  )(jnp.reshape(degp, (NC * RG, 128)), xwg)

  agg1 = _aggregate(jnp.reshape(y1g, (NPAD, HP)), edge_index)

  w2bd = jnp.kron(jnp.eye(8, dtype=jnp.float32),
                  jnp.pad(W2, ((0, 0), (0, HP - H2))))
  b1g = jnp.tile(b1, 8).reshape(1, 128)
  y2g = pl.pallas_call(
      _tc2_body,
      out_shape=jax.ShapeDtypeStruct((RG, 128), jnp.float32),
  )(dinvg, jnp.reshape(agg1, (NC * RG, 128)), w2bd, b1g)

  agg2 = _aggregate(jnp.reshape(y2g, (NPAD, HP)), edge_index)

  b2g = jnp.tile(jnp.pad(b2, (0, HP - H2)), 8).reshape(1, 128)
  bt = jnp.concatenate(
      [batch, jnp.full((NPAD - N,), -1, jnp.int32)]).reshape(RG, 8).T
  wlinp = jnp.pad(Wlin, ((0, HP - H2), (0, 0)))
  out = pl.pallas_call(
      _tc3_body,
      out_shape=jax.ShapeDtypeStruct((G, C), jnp.float32),
  )(dinvg, jnp.reshape(agg2, (NC * RG, 128)), b2g, bt, wlinp,
    blin.reshape(1, C))
  return out


# speedup vs baseline: 117.9629x; 1.1385x over previous
import jax
import jax.numpy as jnp
from jax import lax
from jax.experimental import pallas as pl
from jax.experimental.pallas import tpu as pltpu
from jax.experimental.pallas import tpu_sc as plsc

NC = 2
NS = 16
NW = NC * NS
L = 16

N = 10000
E = 320000
G = 64
HP = 16

NPAD = 10240
RZ = NPAD // NS
EW = E // NW
EB = 2000
NCH = EW // EB
RG = NPAD * HP // 128


def _sc_mesh():
  return plsc.VectorSubcoreMesh(
      core_axis_name="c", subcore_axis_name="s", num_cores=NC,
      num_subcores=NS)


def _deg_kernel(ei_hbm, out_hbm, dacc, idx_v, ones_v, dvm, zrows):
  c = lax.axis_index("c")
  s = lax.axis_index("s")
  w = c * NS + s

  def _fillz(i, carry):
    dvm[pl.ds(i * L, L)] = jnp.zeros((L,), jnp.float32)
    return carry

  def _fill1(i, carry):
    ones_v[pl.ds(i * L, L)] = jnp.full((L,), 1.0, jnp.float32)
    return carry

  lax.fori_loop(0, RZ // L, _fillz, 0)
  lax.fori_loop(0, EB // L, _fill1, 0)
  pltpu.sync_copy(dvm, dacc.at[pl.ds(s * RZ, RZ)])
  plsc.subcore_barrier()

  for i in range(NCH):
    off = w * EW + i * EB
    pltpu.sync_copy(ei_hbm.at[1, pl.ds(off, EB)], idx_v)
    pltpu.sync_copy(ones_v, dacc.at[idx_v], add=True)

  plsc.subcore_barrier()
  pltpu.sync_copy(dacc.at[pl.ds(s * RZ, RZ)], dvm)

  def _expand(g, carry):
    d16 = dvm[pl.ds(g * L, L)]
    for j in range(L):
      zrows[g * L + j, :] = jnp.full((L,), d16[j], jnp.float32)
    return carry

  lax.fori_loop(0, RZ // L, _expand, 0)
  pltpu.sync_copy(zrows, out_hbm.at[c, pl.ds(s * RZ, RZ)])


def _degrees(ei):
  kern = pl.kernel(
      _deg_kernel,
      out_type=jax.ShapeDtypeStruct((NC, NPAD, HP), jnp.float32),
      mesh=_sc_mesh(),
      scratch_types=[
          pltpu.VMEM_SHARED((NPAD,), jnp.float32),
          pltpu.VMEM((EB,), jnp.int32),
          pltpu.VMEM((EB,), jnp.float32),
          pltpu.VMEM((RZ,), jnp.float32),
          pltpu.VMEM((RZ, HP), jnp.float32),
      ],
      compiler_params=pltpu.CompilerParams(use_tc_tiling_on_sc=False),
  )
  return kern(ei)


def _agg_kernel(y_hbm, ei_hbm, out_hbm, ytab, acc, sidx0, sidx1, didx0,
                didx1, rows0, rows1, yrows, semd0, semd1, semg0, semg1,
                sems0, sems1, semi0, semi1):
  c = lax.axis_index("c")
  s = lax.axis_index("s")
  w = c * NS + s
  r0 = s * RZ
  sidx = [sidx0, sidx1]
  didx = [didx0, didx1]
  rows = [rows0, rows1]
  semd = [semd0, semd1]
  semg = [semg0, semg1]
  sems = [sems0, sems1]
  semi = [semi0, semi1]

  pltpu.sync_copy(ei_hbm.at[0, pl.ds(w * EW, EB)], sidx0)
  pltpu.sync_copy(ei_hbm.at[1, pl.ds(w * EW, EB)], didx0)
  pltpu.sync_copy(y_hbm.at[pl.ds(r0, RZ)], yrows)
  pltpu.sync_copy(yrows, ytab.at[pl.ds(r0, RZ)])

  @pl.when(c == 0)
  def _():
    pltpu.sync_copy(yrows, acc.at[pl.ds(r0, RZ)])

  @pl.when(c != 0)
  def _():
    def _fill(i, carry):
      rows0[i, :] = jnp.zeros((L,), jnp.float32)
      return carry

    lax.fori_loop(0, RZ, _fill, 0)
    pltpu.sync_copy(rows0.at[pl.ds(0, RZ)], acc.at[pl.ds(r0, RZ)])

  plsc.subcore_barrier()

  gather_cp = [None] * NCH
  scatter_cp = [None] * NCH
  didx_cp = [None] * NCH
  sidx_cp = [None] * NCH
  gather_cp[0] = pltpu.async_copy(ytab.at[sidx0], rows[0], semg[0])
  for i in range(NCH):
    b = i % 2
    nb = (i + 1) % 2
    if i + 1 < NCH:
      if i > 0:
        scatter_cp[i - 1].wait()
      didx_cp[i + 1] = pltpu.async_copy(
          ei_hbm.at[1, pl.ds(w * EW + (i + 1) * EB, EB)], didx[nb], semd[nb])
      sidx_cp[i + 1] = pltpu.async_copy(
          ei_hbm.at[0, pl.ds(w * EW + (i + 1) * EB, EB)], sidx[nb], semi[nb])
    gather_cp[i].wait()
    if i > 0:
      didx_cp[i].wait()
    scatter_cp[i] = pltpu.async_copy(
        rows[b], acc.at[didx[b]], sems[b], add=True)
    if i + 1 < NCH:
      sidx_cp[i + 1].wait()
      gather_cp[i + 1] = pltpu.async_copy(
          ytab.at[sidx[nb]], rows[nb], semg[nb])
  scatter_cp[NCH - 2].wait()
  scatter_cp[NCH - 1].wait()

  plsc.subcore_barrier()
  pltpu.sync_copy(acc.at[pl.ds(r0, RZ)], yrows)
  pltpu.sync_copy(yrows, out_hbm.at[c, pl.ds(r0, RZ)])


def _aggregate(y, ei):
  kern = pl.kernel(
      _agg_kernel,
      out_type=jax.ShapeDtypeStruct((NC, NPAD, HP), jnp.float32),
      mesh=_sc_mesh(),
      scratch_types=[
          pltpu.VMEM_SHARED((NPAD, HP), jnp.float32),
          pltpu.VMEM_SHARED((NPAD, HP), jnp.float32),
          pltpu.VMEM((EB,), jnp.int32),
          pltpu.VMEM((EB,), jnp.int32),
          pltpu.VMEM((EB,), jnp.int32),
          pltpu.VMEM((EB,), jnp.int32),
          pltpu.VMEM((EB, HP), jnp.float32),
          pltpu.VMEM((EB, HP), jnp.float32),
          pltpu.VMEM((RZ, HP), jnp.float32),
          pltpu.SemaphoreType.DMA,
          pltpu.SemaphoreType.DMA,
          pltpu.SemaphoreType.DMA,
          pltpu.SemaphoreType.DMA,
          pltpu.SemaphoreType.DMA,
          pltpu.SemaphoreType.DMA,
          pltpu.SemaphoreType.DMA,
          pltpu.SemaphoreType.DMA,
      ],
      compiler_params=pltpu.CompilerParams(use_tc_tiling_on_sc=False),
  )
  return kern(y, ei)


def _tc1a_body(x_ref, w1_ref, xw_ref):
  xv = x_ref[:].reshape(N // 8, 8, 128)
  parts = [
      jnp.dot(xv[:, j, :], w1_ref[:], preferred_element_type=jnp.float32)
      for j in range(8)
  ]
  xw_ref[0:N // 8, :] = jnp.concatenate(parts, axis=1)
  xw_ref[N // 8:, :] = jnp.zeros((RG - N // 8, 128), jnp.float32)


def _tc1b_body(degr_ref, xw_ref, y1_ref, dinv_ref):
  dinv = lax.rsqrt(degr_ref[0:RG] + degr_ref[RG:2 * RG] + 1.0)
  dinv_ref[:] = dinv
  y1_ref[:] = xw_ref[:] * dinv


def _tc2_body(dinv_ref, agg1r_ref, w2bd_ref, b1g_ref, y2_ref):
  dinv = dinv_ref[:]
  sfull = agg1r_ref[0:RG] + agg1r_ref[RG:2 * RG]
  h1 = jax.nn.relu(dinv * sfull + b1g_ref[:])
  y2_ref[:] = jnp.dot(h1, w2bd_ref[:],
                      preferred_element_type=jnp.float32) * dinv


def _tc3_body(dinv_ref, agg2r_ref, b2g_ref, bt_ref, wlin_ref, blin_ref,
              out_ref):
  h2 = jax.nn.relu(
      dinv_ref[:] * (agg2r_ref[0:RG] + agg2r_ref[RG:2 * RG]) + b2g_ref[:])
  bt = bt_ref[:]
  gid = lax.broadcasted_iota(jnp.int32, (G, 1), 0)
  pooled = jnp.zeros((G, HP), jnp.float32)
  cnt = jnp.zeros((G, 1), jnp.float32)
  for j in range(8):
    mj = (bt[j:j + 1, :] == gid).astype(jnp.float32)
    pj = jnp.dot(mj, h2, preferred_element_type=jnp.float32)
    pooled = pooled + pj[:, HP * j:HP * (j + 1)]
    cnt = cnt + jnp.sum(mj, axis=1, keepdims=True)
  pooled = pooled / jnp.maximum(cnt, 1.0)
  out_ref[:] = jnp.dot(pooled, wlin_ref[:],
                       preferred_element_type=jnp.float32) + blin_ref[:]


def kernel(x, edge_index, batch, W1, b1, W2, b2, Wlin, blin):
  C = Wlin.shape[1]
  H2 = W2.shape[1]

  degp = _degrees(edge_index)

  xwg = pl.pallas_call(
      _tc1a_body,
      out_shape=jax.ShapeDtypeStruct((RG, 128), jnp.float32),
  )(x, W1)

  y1g, dinvg = pl.pallas_call(
      _tc1b_body,
      out_shape=[
          jax.ShapeDtypeStruct((RG, 128), jnp.float32),
          jax.ShapeDtypeStruct((RG, 128), jnp.float32),
      ],
  )(jnp.reshape(degp, (NC * RG, 128)), xwg)

  agg1 = _aggregate(jnp.reshape(y1g, (NPAD, HP)), edge_index)

  w2bd = jnp.kron(jnp.eye(8, dtype=jnp.float32),
                  jnp.pad(W2, ((0, 0), (0, HP - H2))))
  b1g = jnp.tile(b1, 8).reshape(1, 128)
  y2g = pl.pallas_call(
      _tc2_body,
      out_shape=jax.ShapeDtypeStruct((RG, 128), jnp.float32),
  )(dinvg, jnp.reshape(agg1, (NC * RG, 128)), w2bd, b1g)

  agg2 = _aggregate(jnp.reshape(y2g, (NPAD, HP)), edge_index)

  b2g = jnp.tile(jnp.pad(b2, (0, HP - H2)), 8).reshape(1, 128)
  bt = jnp.concatenate(
      [batch, jnp.full((NPAD - N,), -1, jnp.int32)]).reshape(RG, 8).T
  wlinp = jnp.pad(Wlin, ((0, HP - H2), (0, 0)))
  out = pl.pallas_call(
      _tc3_body,
      out_shape=jax.ShapeDtypeStruct((G, C), jnp.float32),
  )(dinvg, jnp.reshape(agg2, (NC * RG, 128)), b2g, bt, wlinp,
    blin.reshape(1, C))
  return out
